# baseline (speedup 1.0000x reference)
import functools

import jax
import jax.numpy as jnp
from jax import lax
from jax.experimental import pallas as pl
from jax.experimental.pallas import tpu as pltpu
from jax.experimental.pallas import tpu_sc as plsc

_V = 100000
_E = 200
_H = 32
_G = 4 * _H
_L = 50
_N = 4096

_ROW_BLK = 10000
_K = 5
_LP = _L // _K

_NC, _NS = 2, 16
_NW = _NC * _NS
_TOT = _L * _N
_TOTP = _TOT // _K
_RPW = _TOTP // _NW
_CH = _RPW // 128


def _proj_body(emb_ref, wih_ref, out_ref):
    out_ref[...] = lax.dot_general(
        emb_ref[...], wih_ref[...], (((1,), (1,)), ((), ())),
        preferred_element_type=jnp.float32)


def _project_table(emb, W_ih):
    return pl.pallas_call(
        _proj_body,
        grid=(_V // _ROW_BLK,),
        in_specs=[
            pl.BlockSpec((_ROW_BLK, _E), lambda i: (i, 0)),
            pl.BlockSpec((_G, _E), lambda i: (0, 0)),
        ],
        out_specs=pl.BlockSpec((_ROW_BLK, _G), lambda i: (i, 0)),
        out_shape=jax.ShapeDtypeStruct((_V, _G), jnp.float32),
    )(emb, W_ih)


def _gather_body(p_hbm, x_hbm, out_hbm, idx_v, rows_v, s0, s1):
    wid = lax.axis_index("s") * _NC + lax.axis_index("c")
    pltpu.sync_copy(x_hbm.at[wid], idx_v)
    base = wid * _RPW

    def start(r, buf, sem):
        pltpu.async_copy(p_hbm.at[idx_v.at[r]], rows_v.at[buf], sem)

    def drain(r, buf, sem):
        pltpu.make_async_copy(p_hbm.at[idx_v.at[r]], rows_v.at[buf], sem).wait()
        pltpu.sync_copy(rows_v.at[buf], out_hbm.at[pl.ds(base + r * 128, 128)])

    start(0, 0, s0)

    def body(pr, carry):
        r0 = pr * 2
        r1 = r0 + 1
        start(r1, 1, s1)
        drain(r0, 0, s0)

        @pl.when(r1 + 1 < _CH)
        def _():
            start(r1 + 1, 0, s0)

        drain(r1, 1, s1)
        return carry

    lax.fori_loop(0, _CH // 2, body, 0)


def _gather(p, x3):
    mesh = plsc.VectorSubcoreMesh(core_axis_name="c", subcore_axis_name="s")
    fn = functools.partial(
        pl.kernel,
        mesh=mesh,
        out_type=jax.ShapeDtypeStruct((_TOTP, _G), jnp.float32),
        scratch_types=[
            pltpu.VMEM((_CH, 128), jnp.int32),
            pltpu.VMEM((2, 128, _G), jnp.float32),
            pltpu.SemaphoreType.DMA,
            pltpu.SemaphoreType.DMA,
        ],
    )(_gather_body)
    return fn(p, x3)


def _lstm_body(g_ref, whh_ref, wl_ref, b_ref, bl_ref, h0_ref, c0_ref,
               out_ref, hn_ref, cn_ref, h_ref, c_ref):
    t = pl.program_id(0)

    @pl.when(t == 0)
    def _():
        h_ref[...] = h0_ref[...]
        c_ref[...] = c0_ref[...]

    h = h_ref[...]
    grec = lax.dot_general(
        h, whh_ref[...], (((1,), (1,)), ((), ())),
        preferred_element_type=jnp.float32)
    gates = g_ref[0] + grec + b_ref[...]
    i = jax.nn.sigmoid(gates[:, 0:_H])
    f = jax.nn.sigmoid(gates[:, _H:2 * _H])
    g = jnp.tanh(gates[:, 2 * _H:3 * _H])
    o = jax.nn.sigmoid(gates[:, 3 * _H:4 * _H])
    c = f * c_ref[...] + i * g
    h2 = o * jnp.tanh(c)
    c_ref[...] = c
    h_ref[...] = h2
    hn_ref[...] = h2
    cn_ref[...] = c
    logits = lax.dot_general(
        wl_ref[...], h2, (((1,), (1,)), ((), ())),
        preferred_element_type=jnp.float32) + bl_ref[...]
    m = jnp.max(logits, axis=1, keepdims=True)
    lse = m + jnp.log(jnp.sum(jnp.exp(logits - m), axis=1, keepdims=True))
    out_ref[0] = logits - lse


def _lstm_phase(g, W_hh, W_l, b, b_l, h0, c0):
    return pl.pallas_call(
        _lstm_body,
        grid=(_LP,),
        in_specs=[
            pl.BlockSpec((1, _N, _G), lambda t: (t, 0, 0)),
            pl.BlockSpec((_G, _H), lambda t: (0, 0)),
            pl.BlockSpec((2, _H), lambda t: (0, 0)),
            pl.BlockSpec((1, _G), lambda t: (0, 0)),
            pl.BlockSpec((2, 1), lambda t: (0, 0)),
            pl.BlockSpec((_N, _H), lambda t: (0, 0)),
            pl.BlockSpec((_N, _H), lambda t: (0, 0)),
        ],
        out_specs=[
            pl.BlockSpec((1, 2, _N), lambda t: (t, 0, 0)),
            pl.BlockSpec((_N, _H), lambda t: (0, 0)),
            pl.BlockSpec((_N, _H), lambda t: (0, 0)),
        ],
        out_shape=[
            jax.ShapeDtypeStruct((_LP, 2, _N), jnp.float32),
            jax.ShapeDtypeStruct((_N, _H), jnp.float32),
            jax.ShapeDtypeStruct((_N, _H), jnp.float32),
        ],
        scratch_shapes=[
            pltpu.VMEM((_N, _H), jnp.float32),
            pltpu.VMEM((_N, _H), jnp.float32),
        ],
    )(g, W_hh, W_l, b, b_l, h0, c0)


def kernel(x, emb, W_ih, W_hh, b_ih, b_hh, W_l, b_l):
    p = _project_table(emb, W_ih)
    xf = x.reshape(_K, _NW, _CH, 128)
    gs = [_gather(p, xf[k]) for k in range(_K)]
    b = (b_ih + b_hh).reshape(1, _G)
    bl = b_l.reshape(2, 1)
    h = jnp.zeros((_N, _H), jnp.float32)
    c = jnp.zeros((_N, _H), jnp.float32)
    outs = []
    for k in range(_K):
        g = gs[k].reshape(_LP, _N, _G)
        out_k, h, c = _lstm_phase(g, W_hh, W_l, b, bl, h, c)
        outs.append(out_k)
    out_t = jnp.concatenate(outs, axis=0)
    return jnp.swapaxes(out_t, 1, 2)


# speedup vs baseline: 1.3082x; 1.3082x over previous
import functools

import jax
import jax.numpy as jnp
from jax import lax
from jax.experimental import pallas as pl
from jax.experimental.pallas import tpu as pltpu
from jax.experimental.pallas import tpu_sc as plsc

_V = 100000
_E = 200
_H = 32
_G = 4 * _H
_L = 50
_N = 4096

_ROW_BLK = 10000
_K = 5
_LP = _L // _K

_NC, _NS = 2, 16
_NW = _NC * _NS
_TOT = _L * _N
_TOTP = _TOT // _K
_RPW = _TOTP // _NW
_CH = _RPW // 128


def _proj_body(emb_ref, wih_ref, out_ref):
    out_ref[...] = lax.dot_general(
        emb_ref[...], wih_ref[...], (((1,), (1,)), ((), ())),
        preferred_element_type=jnp.float32)


def _project_table(emb, W_ih):
    return pl.pallas_call(
        _proj_body,
        grid=(_V // _ROW_BLK,),
        in_specs=[
            pl.BlockSpec((_ROW_BLK, _E), lambda i: (i, 0)),
            pl.BlockSpec((_G, _E), lambda i: (0, 0)),
        ],
        out_specs=pl.BlockSpec((_ROW_BLK, _G), lambda i: (i, 0)),
        out_shape=jax.ShapeDtypeStruct((_V, _G), jnp.float32),
    )(emb, W_ih)


_D = 4


def _gather_body(p_hbm, x_hbm, out_hbm, idx_v, rows_v, *sems):
    wid = lax.axis_index("s") * _NC + lax.axis_index("c")
    pltpu.sync_copy(x_hbm.at[wid], idx_v)
    base = wid * _RPW
    gsem = sems[:_D]
    wsem = sems[_D:]

    def gstart(r):
        pltpu.async_copy(p_hbm.at[idx_v.at[r]], rows_v.at[r % _D], gsem[r % _D])

    def gwait(r):
        pltpu.make_async_copy(
            p_hbm.at[idx_v.at[r]], rows_v.at[r % _D], gsem[r % _D]).wait()

    def wstart(r):
        pltpu.async_copy(
            rows_v.at[r % _D], out_hbm.at[pl.ds(base + r * 128, 128)],
            wsem[r % _D])

    def wwait(r):
        pltpu.make_async_copy(
            rows_v.at[r % _D], out_hbm.at[pl.ds(base + r * 128, 128)],
            wsem[r % _D]).wait()

    for r in range(min(_D, _CH)):
        gstart(r)
    for r in range(_CH):
        gwait(r)
        wstart(r)
        if r + _D < _CH:
            wwait(r)
            gstart(r + _D)
    for r in range(max(0, _CH - _D), _CH):
        wwait(r)


def _gather(p, x3):
    mesh = plsc.VectorSubcoreMesh(core_axis_name="c", subcore_axis_name="s")
    fn = functools.partial(
        pl.kernel,
        mesh=mesh,
        out_type=jax.ShapeDtypeStruct((_TOTP, _G), jnp.float32),
        scratch_types=[
            pltpu.VMEM((_CH, 128), jnp.int32),
            pltpu.VMEM((_D, 128, _G), jnp.float32),
        ] + [pltpu.SemaphoreType.DMA] * (2 * _D),
    )(_gather_body)
    return fn(p, x3)


def _lstm_body(g_ref, eye_ref, whh_ref, wl_ref, b_ref, bl_ref, h0_ref, c0_ref,
               out_ref, hn_ref, cn_ref, h_ref, c_ref):
    t = pl.program_id(0)

    @pl.when(t == 0)
    def _():
        h_ref[...] = h0_ref[...]
        c_ref[...] = c0_ref[...]

    h = h_ref[...]
    gin = lax.dot_general(
        eye_ref[...], g_ref[0], (((1,), (1,)), ((), ())),
        preferred_element_type=jnp.float32)
    grec = lax.dot_general(
        whh_ref[...], h, (((1,), (0,)), ((), ())),
        preferred_element_type=jnp.float32)
    gates = gin + grec + b_ref[...]
    i = jax.nn.sigmoid(gates[0:_H])
    f = jax.nn.sigmoid(gates[_H:2 * _H])
    g = jnp.tanh(gates[2 * _H:3 * _H])
    o = jax.nn.sigmoid(gates[3 * _H:4 * _H])
    c = f * c_ref[...] + i * g
    h2 = o * jnp.tanh(c)
    c_ref[...] = c
    h_ref[...] = h2
    hn_ref[...] = h2
    cn_ref[...] = c
    logits = lax.dot_general(
        wl_ref[...], h2, (((1,), (0,)), ((), ())),
        preferred_element_type=jnp.float32) + bl_ref[...]
    m = jnp.max(logits, axis=1, keepdims=True)
    lse = m + jnp.log(jnp.sum(jnp.exp(logits - m), axis=1, keepdims=True))
    out_ref[0] = logits - lse


def _lstm_phase(g, eye, W_hh, W_l, b, b_l, h0, c0):
    return pl.pallas_call(
        _lstm_body,
        grid=(_LP,),
        in_specs=[
            pl.BlockSpec((1, _N, _G), lambda t: (t, 0, 0)),
            pl.BlockSpec((_G, _G), lambda t: (0, 0)),
            pl.BlockSpec((_G, _H), lambda t: (0, 0)),
            pl.BlockSpec((2, _H), lambda t: (0, 0)),
            pl.BlockSpec((_G, 1), lambda t: (0, 0)),
            pl.BlockSpec((2, 1), lambda t: (0, 0)),
            pl.BlockSpec((_H, _N), lambda t: (0, 0)),
            pl.BlockSpec((_H, _N), lambda t: (0, 0)),
        ],
        out_specs=[
            pl.BlockSpec((1, 2, _N), lambda t: (t, 0, 0)),
            pl.BlockSpec((_H, _N), lambda t: (0, 0)),
            pl.BlockSpec((_H, _N), lambda t: (0, 0)),
        ],
        out_shape=[
            jax.ShapeDtypeStruct((_LP, 2, _N), jnp.float32),
            jax.ShapeDtypeStruct((_H, _N), jnp.float32),
            jax.ShapeDtypeStruct((_H, _N), jnp.float32),
        ],
        scratch_shapes=[
            pltpu.VMEM((_H, _N), jnp.float32),
            pltpu.VMEM((_H, _N), jnp.float32),
        ],
    )(g, eye, W_hh, W_l, b, b_l, h0, c0)


def kernel(x, emb, W_ih, W_hh, b_ih, b_hh, W_l, b_l):
    p = _project_table(emb, W_ih)
    xf = x.reshape(_K, _NW, _CH, 128)
    gs = [_gather(p, xf[k]) for k in range(_K)]
    eye = jnp.eye(_G, dtype=jnp.float32)
    b = (b_ih + b_hh).reshape(_G, 1)
    bl = b_l.reshape(2, 1)
    h = jnp.zeros((_H, _N), jnp.float32)
    c = jnp.zeros((_H, _N), jnp.float32)
    outs = []
    for k in range(_K):
        g = gs[k].reshape(_LP, _N, _G)
        out_k, h, c = _lstm_phase(g, eye, W_hh, W_l, b, bl, h, c)
        outs.append(out_k)
    out_t = jnp.concatenate(outs, axis=0)
    return jnp.swapaxes(out_t, 1, 2)


# speedup vs baseline: 1.5787x; 1.2067x over previous
import functools

import jax
import jax.numpy as jnp
from jax import lax
from jax.experimental import pallas as pl
from jax.experimental.pallas import tpu as pltpu
from jax.experimental.pallas import tpu_sc as plsc

_V = 100000
_E = 200
_H = 32
_G = 4 * _H
_L = 50
_N = 4096

_ROW_BLK = 10000
_K = 5
_LP = _L // _K

_NC, _NS = 2, 16
_NW = _NC * _NS
_TOT = _L * _N
_TOTP = _TOT // _K
_RPW = _TOTP // _NW
_CH = _RPW // 128


_CW = 1280
_CHUNKS = [(i * _CW, _CW) for i in range(_V // _CW)] + [(_V - _V % _CW, 128)]
_VT = _V - _V % 128


def _proj_body(embT_hbm, wihT_ref, tail_ref, out_hbm, ib, ob, i0, i1, o0, o1):
    isem = (i0, i1)
    osem = (o0, o1)

    def istart(j):
        c, w = _CHUNKS[j]
        pltpu.make_async_copy(
            embT_hbm.at[:, pl.ds(c, w)], ib.at[j % 2, :, pl.ds(0, w)],
            isem[j % 2]).start()

    def iwait(j):
        c, w = _CHUNKS[j]
        pltpu.make_async_copy(
            embT_hbm.at[:, pl.ds(c, w)], ib.at[j % 2, :, pl.ds(0, w)],
            isem[j % 2]).wait()

    def ostart(j):
        c, w = _CHUNKS[j]
        pltpu.make_async_copy(
            ob.at[j % 2, pl.ds(0, w), :], out_hbm.at[pl.ds(c, w)],
            osem[j % 2]).start()

    def owait(j):
        c, w = _CHUNKS[j]
        pltpu.make_async_copy(
            ob.at[j % 2, pl.ds(0, w), :], out_hbm.at[pl.ds(c, w)],
            osem[j % 2]).wait()

    n = len(_CHUNKS)
    istart(0)
    istart(1)
    for j in range(n):
        iwait(j)
        if j >= 2:
            owait(j - 2)
        c, w = _CHUNKS[j]
        ob[j % 2, pl.ds(0, w), :] = lax.dot_general(
            ib[j % 2, :, pl.ds(0, w)], wihT_ref[...],
            (((0,), (0,)), ((), ())), preferred_element_type=jnp.float32)
        ostart(j)
        if j + 2 < n:
            istart(j + 2)
    owait(n - 2)
    owait(n - 1)
    ob[0, pl.ds(0, 32), :] = lax.dot_general(
        tail_ref[...], wihT_ref[...], (((0,), (0,)), ((), ())),
        preferred_element_type=jnp.float32)
    pltpu.sync_copy(ob.at[0, pl.ds(0, 32), :], out_hbm.at[pl.ds(_VT, 32)])


def _project_table(embT, W_ihT):
    tail = lax.slice(embT, (0, _VT), (_E, _V))
    return pl.pallas_call(
        _proj_body,
        in_specs=[
            pl.BlockSpec(memory_space=pltpu.MemorySpace.HBM),
            pl.BlockSpec(memory_space=pltpu.MemorySpace.VMEM),
            pl.BlockSpec(memory_space=pltpu.MemorySpace.VMEM),
        ],
        out_specs=pl.BlockSpec(memory_space=pltpu.MemorySpace.HBM),
        out_shape=jax.ShapeDtypeStruct((_V, _G), jnp.float32),
        scratch_shapes=[
            pltpu.VMEM((2, _E, _CW), jnp.float32),
            pltpu.VMEM((2, _CW, _G), jnp.float32),
            pltpu.SemaphoreType.DMA,
            pltpu.SemaphoreType.DMA,
            pltpu.SemaphoreType.DMA,
            pltpu.SemaphoreType.DMA,
        ],
    )(embT, W_ihT, tail)


_D = 4


def _gather_body(p_hbm, x_hbm, out_hbm, idx_v, rows_v, *sems):
    wid = lax.axis_index("s") * _NC + lax.axis_index("c")
    pltpu.sync_copy(x_hbm.at[wid], idx_v)
    base = wid * _RPW
    gsem = sems[:_D]
    wsem = sems[_D:]

    def gstart(r):
        pltpu.async_copy(p_hbm.at[idx_v.at[r]], rows_v.at[r % _D], gsem[r % _D])

    def gwait(r):
        pltpu.make_async_copy(
            p_hbm.at[idx_v.at[r]], rows_v.at[r % _D], gsem[r % _D]).wait()

    def wstart(r):
        pltpu.async_copy(
            rows_v.at[r % _D], out_hbm.at[pl.ds(base + r * 128, 128)],
            wsem[r % _D])

    def wwait(r):
        pltpu.make_async_copy(
            rows_v.at[r % _D], out_hbm.at[pl.ds(base + r * 128, 128)],
            wsem[r % _D]).wait()

    for r in range(min(_D, _CH)):
        gstart(r)
    for r in range(_CH):
        gwait(r)
        wstart(r)
        if r + _D < _CH:
            wwait(r)
            gstart(r + _D)
    for r in range(max(0, _CH - _D), _CH):
        wwait(r)


def _gather(p, x3):
    mesh = plsc.VectorSubcoreMesh(core_axis_name="c", subcore_axis_name="s")
    fn = functools.partial(
        pl.kernel,
        mesh=mesh,
        out_type=jax.ShapeDtypeStruct((_TOTP, _G), jnp.float32),
        scratch_types=[
            pltpu.VMEM((_CH, 128), jnp.int32),
            pltpu.VMEM((_D, 128, _G), jnp.float32),
        ] + [pltpu.SemaphoreType.DMA] * (2 * _D),
    )(_gather_body)
    return fn(p, x3)


def _lstm_body(g_ref, eye_ref, whh_ref, wl_ref, b_ref, bl_ref, h0_ref, c0_ref,
               out_ref, hn_ref, cn_ref, h_ref, c_ref):
    t = pl.program_id(0)

    @pl.when(t == 0)
    def _():
        h_ref[...] = h0_ref[...]
        c_ref[...] = c0_ref[...]

    h = h_ref[...]
    gin = lax.dot_general(
        eye_ref[...], g_ref[0], (((1,), (1,)), ((), ())),
        preferred_element_type=jnp.float32)
    grec = lax.dot_general(
        whh_ref[...], h, (((1,), (0,)), ((), ())),
        preferred_element_type=jnp.float32)
    gates = gin + grec + b_ref[...]
    i = jax.nn.sigmoid(gates[0:_H])
    f = jax.nn.sigmoid(gates[_H:2 * _H])
    g = jnp.tanh(gates[2 * _H:3 * _H])
    o = jax.nn.sigmoid(gates[3 * _H:4 * _H])
    c = f * c_ref[...] + i * g
    h2 = o * jnp.tanh(c)
    c_ref[...] = c
    h_ref[...] = h2
    hn_ref[...] = h2
    cn_ref[...] = c
    logits = lax.dot_general(
        wl_ref[...], h2, (((1,), (0,)), ((), ())),
        preferred_element_type=jnp.float32) + bl_ref[...]
    m = jnp.max(logits, axis=1, keepdims=True)
    lse = m + jnp.log(jnp.sum(jnp.exp(logits - m), axis=1, keepdims=True))
    out_ref[0] = logits - lse


def _lstm_phase(g, eye, W_hh, W_l, b, b_l, h0, c0):
    return pl.pallas_call(
        _lstm_body,
        grid=(_LP,),
        in_specs=[
            pl.BlockSpec((1, _N, _G), lambda t: (t, 0, 0)),
            pl.BlockSpec((_G, _G), lambda t: (0, 0)),
            pl.BlockSpec((_G, _H), lambda t: (0, 0)),
            pl.BlockSpec((2, _H), lambda t: (0, 0)),
            pl.BlockSpec((_G, 1), lambda t: (0, 0)),
            pl.BlockSpec((2, 1), lambda t: (0, 0)),
            pl.BlockSpec((_H, _N), lambda t: (0, 0)),
            pl.BlockSpec((_H, _N), lambda t: (0, 0)),
        ],
        out_specs=[
            pl.BlockSpec((1, 2, _N), lambda t: (t, 0, 0)),
            pl.BlockSpec((_H, _N), lambda t: (0, 0)),
            pl.BlockSpec((_H, _N), lambda t: (0, 0)),
        ],
        out_shape=[
            jax.ShapeDtypeStruct((_LP, 2, _N), jnp.float32),
            jax.ShapeDtypeStruct((_H, _N), jnp.float32),
            jax.ShapeDtypeStruct((_H, _N), jnp.float32),
        ],
        scratch_shapes=[
            pltpu.VMEM((_H, _N), jnp.float32),
            pltpu.VMEM((_H, _N), jnp.float32),
        ],
    )(g, eye, W_hh, W_l, b, b_l, h0, c0)


def kernel(x, emb, W_ih, W_hh, b_ih, b_hh, W_l, b_l):
    p = _project_table(emb.T, W_ih.T)
    xf = x.reshape(_K, _NW, _CH, 128)
    gs = [_gather(p, xf[k]) for k in range(_K)]
    eye = jnp.eye(_G, dtype=jnp.float32)
    b = (b_ih + b_hh).reshape(_G, 1)
    bl = b_l.reshape(2, 1)
    h = jnp.zeros((_H, _N), jnp.float32)
    c = jnp.zeros((_H, _N), jnp.float32)
    outs = []
    for k in range(_K):
        g = gs[k].reshape(_LP, _N, _G)
        out_k, h, c = _lstm_phase(g, eye, W_hh, W_l, b, bl, h, c)
        outs.append(out_k)
    out_t = jnp.concatenate(outs, axis=0)
    return jnp.swapaxes(out_t, 1, 2)
